# baseline (speedup 1.0000x reference)
import jax
import jax.numpy as jnp
from jax import lax
from jax.experimental import pallas as pl
from jax.experimental.pallas import tpu as pltpu
from jax.experimental.pallas import tpu_sc as plsc

MAP_CELLS = 400
N_BINS = MAP_CELLS * MAP_CELLS
DUMP = N_BINS
BINS_PAD = 160256
ZSLICE = BINS_PAD // 16
OSLICE = N_BINS // 16

WR0 = 120
WRN = 160
WC0 = 128
WCN = 144
WSZ = WRN * WCN

NW = 32
CHUNK = 2048
GROUPS = CHUNK // 16
N_PTS = 1_000_000
FULL_CHUNKS = N_PTS // CHUNK
TAIL = N_PTS - FULL_CHUNKS * CHUNK
HI_W = FULL_CHUNKS - 15 * NW

MAGIC = 12582912.0
CELL = 0.1
SHIFT = 200.0

OUT_WORDS = 2 * N_BINS + NW * WSZ


def _sc_body(ph, th, outg, outw, cb0, cb1, idxrow, onesv, wbuf, obuf, hist,
             sem0, sem1):
    c = lax.axis_index("c")
    s = lax.axis_index("s")
    wid = c * 16 + s
    cbase = jnp.where(wid < HI_W, wid * 16, HI_W + wid * 15)
    ntrips = jnp.where(wid < HI_W, 16, 15)

    ones16 = jnp.ones((16,), jnp.float32)
    zeros16 = jnp.zeros((16,), jnp.float32)
    i16 = lax.iota(jnp.int32, 16)
    c0 = jnp.zeros((16,), jnp.int32)
    c1 = c0 + 1
    c2 = c0 + 2
    for i in range(8):
        onesv[pl.ds(16 * i, 16)] = ones16

    def _zero(i, carry):
        wbuf[pl.ds(i * 16, 16)] = zeros16
        return carry

    lax.fori_loop(0, WSZ // 16, _zero, None)
    pltpu.sync_copy(wbuf.at[pl.ds(0, ZSLICE)], hist.at[pl.ds(s * ZSLICE, ZSLICE)])
    plsc.subcore_barrier()

    def _cell_rows(xs, zs):
        rr = (zs / CELL + SHIFT + MAGIC) - MAGIC
        rc = (xs / CELL + SHIFT + MAGIC) - MAGIC
        return rr, rc

    def _process(cb):
        def _row(r, rany):
            for j in range(8):
                ridx = i16 + (r * 128 + j * 16)
                xs = plsc.load_gather(cb, [ridx, c0])
                ys = plsc.load_gather(cb, [ridx, c1])
                zs = plsc.load_gather(cb, [ridx, c2])
                rr, rc = _cell_rows(xs, zs)
                masky = (ys < 0.0) & (ys > -1.0)
                inwin = ((rr >= float(WR0)) & (rr < float(WR0 + WRN))
                         & (rc >= float(WC0)) & (rc < float(WC0 + WCN))
                         & masky)
                wif = rr * float(WCN) + rc - float(WR0 * WCN + WC0)
                widx = jnp.where(inwin, wif, 0.0).astype(jnp.int32)
                plsc.addupdate_scatter(wbuf, [widx], ones16, mask=inwin)
                rany = rany | (masky & (~inwin))
            return rany

        rany = lax.fori_loop(0, GROUPS // 8, _row,
                             jnp.zeros((16,), jnp.bool_))
        fire = jnp.any(rany)

        @pl.when(fire)
        def _slow():
            def _srow(r, carry):
                for j in range(8):
                    ridx = i16 + (r * 128 + j * 16)
                    xs = plsc.load_gather(cb, [ridx, c0])
                    ys = plsc.load_gather(cb, [ridx, c1])
                    zs = plsc.load_gather(cb, [ridx, c2])
                    rr, rc = _cell_rows(xs, zs)
                    masky = (ys < 0.0) & (ys > -1.0)
                    inwin = ((rr >= float(WR0)) & (rr < float(WR0 + WRN))
                             & (rc >= float(WC0)) & (rc < float(WC0 + WCN)))
                    inb = ((rr >= 0.0) & (rr <= 399.0)
                           & (rc >= 0.0) & (rc <= 399.0))
                    rest = masky & inb & (~inwin)
                    idxf = jnp.where(rest, rr * 400.0 + rc, float(DUMP))
                    idxrow[0, pl.ds(j * 16, 16)] = idxf.astype(jnp.int32)
                pltpu.sync_copy(onesv, hist.at[idxrow.at[0]], add=True)
                return carry

            lax.fori_loop(0, GROUPS // 8, _srow, None)

    def _fire(k, cb, sem):
        roff = (cbase + k) * CHUNK
        pltpu.async_copy(ph.at[pl.ds(roff, CHUNK)], cb, sem)

    def _wait(cb, sem):
        pltpu.make_async_copy(ph.at[pl.ds(0, CHUNK)], cb, sem).wait()

    _fire(0, cb0, sem0)
    _fire(1, cb1, sem1)

    def _pair(kk, carry):
        k0 = kk * 2
        _wait(cb0, sem0)
        _process(cb0)

        @pl.when(k0 + 2 < ntrips)
        def _f0():
            _fire(k0 + 2, cb0, sem0)

        @pl.when(k0 + 1 < ntrips)
        def _c1():
            _wait(cb1, sem1)
            _process(cb1)

            @pl.when(k0 + 3 < ntrips)
            def _f1():
                _fire(k0 + 3, cb1, sem1)

        return carry

    lax.fori_loop(0, 8, _pair, None)

    @pl.when(wid == NW - 1)
    def _tail():
        pltpu.async_copy(th.at[pl.ds(0, CHUNK)], cb0, sem0)
        pltpu.make_async_copy(th.at[pl.ds(0, CHUNK)], cb0, sem0).wait()
        _process(cb0)

    plsc.subcore_barrier()

    pltpu.sync_copy(wbuf, outw.at[pl.ds(wid * WSZ, WSZ)])
    pltpu.sync_copy(hist.at[pl.ds(s * OSLICE, OSLICE)],
                    obuf.at[pl.ds(0, OSLICE)])
    pltpu.sync_copy(obuf.at[pl.ds(0, OSLICE)],
                    outg.at[pl.ds(c * N_BINS + s * OSLICE, OSLICE)])


_sc_hist = pl.kernel(
    _sc_body,
    out_type=(jax.ShapeDtypeStruct((2 * N_BINS,), jnp.float32),
              jax.ShapeDtypeStruct((NW * WSZ,), jnp.float32)),
    mesh=plsc.VectorSubcoreMesh(core_axis_name="c", subcore_axis_name="s"),
    compiler_params=pltpu.CompilerParams(needs_layout_passes=False,
                                         use_tc_tiling_on_sc=False),
    scratch_types=[
        pltpu.VMEM((CHUNK, 3), jnp.float32),
        pltpu.VMEM((CHUNK, 3), jnp.float32),
        pltpu.VMEM((1, 128), jnp.int32),
        pltpu.VMEM((128,), jnp.float32),
        pltpu.VMEM((WSZ,), jnp.float32),
        pltpu.VMEM((ZSLICE,), jnp.float32),
        pltpu.VMEM_SHARED((BINS_PAD,), jnp.float32),
        pltpu.SemaphoreType.DMA,
        pltpu.SemaphoreType.DMA,
    ],
)


def _final_body(g_ref, w_ref, o_ref):
    o_ref[...] = g_ref[0] + g_ref[1]
    wsum = jnp.sum(w_ref[...], axis=0)
    o_ref[pl.ds(WR0, WRN), pl.ds(WC0, WCN)] = (
        o_ref[pl.ds(WR0, WRN), pl.ds(WC0, WCN)] + wsum)


def kernel(sparse_depth):
    tail = jnp.pad(sparse_depth[FULL_CHUNKS * CHUNK:],
                   ((0, CHUNK - TAIL), (0, 0)))
    og, ow = _sc_hist(sparse_depth, tail)
    g = og.reshape(2, MAP_CELLS, MAP_CELLS)
    w = ow.reshape(NW, WRN, WCN)
    return pl.pallas_call(
        _final_body,
        out_shape=jax.ShapeDtypeStruct((MAP_CELLS, MAP_CELLS), jnp.float32),
    )(g, w)


# speedup vs baseline: 33.5128x; 33.5128x over previous
import jax
import jax.numpy as jnp
from jax import lax
from jax.experimental import pallas as pl
from jax.experimental.pallas import tpu as pltpu
from jax.experimental.pallas import tpu_sc as plsc

MAP_CELLS = 400
N_BINS = MAP_CELLS * MAP_CELLS
DUMP = N_BINS
BINS_PAD = 160256
ZSLICE = BINS_PAD // 16
OSLICE = N_BINS // 16

WR0 = 120
WRN = 160
WC0 = 128
WCN = 144
WSZ = WRN * WCN

NW = 32
CHUNK = 2048
GROUPS = CHUNK // 16
N_CHUNKS = 16
PW = CHUNK * N_CHUNKS
NP = NW * PW

MAGIC = 12582912.0
CELL = 0.1
SHIFT = 200.0


def _sc_body(xh, yh, zh, outg, outw, xv0, yv0, zv0, xv1, yv1, zv1,
             idxrow, onesv, wbuf, obuf, hist, sem0, sem1):
    c = lax.axis_index("c")
    s = lax.axis_index("s")
    wid = c * 16 + s
    base = wid * PW

    ones16 = jnp.ones((16,), jnp.float32)
    zeros16 = jnp.zeros((16,), jnp.float32)
    for i in range(8):
        onesv[pl.ds(16 * i, 16)] = ones16

    def _zero(i, carry):
        wbuf[pl.ds(i * 16, 16)] = zeros16
        return carry

    lax.fori_loop(0, WSZ // 16, _zero, None)
    pltpu.sync_copy(wbuf.at[pl.ds(0, ZSLICE)], hist.at[pl.ds(s * ZSLICE, ZSLICE)])
    plsc.subcore_barrier()

    def _cell_rows(xs, zs):
        rr = (zs / CELL + SHIFT + MAGIC) - MAGIC
        rc = (xs / CELL + SHIFT + MAGIC) - MAGIC
        return rr, rc

    def _process(bufs):
        xv, yv, zv = bufs

        def _row(r, rany):
            for j in range(8):
                o = r * 128 + j * 16
                xs = xv[pl.ds(o, 16)]
                ys = yv[pl.ds(o, 16)]
                zs = zv[pl.ds(o, 16)]
                rr, rc = _cell_rows(xs, zs)
                masky = (ys < 0.0) & (ys > -1.0)
                inwin = ((rr >= float(WR0)) & (rr < float(WR0 + WRN))
                         & (rc >= float(WC0)) & (rc < float(WC0 + WCN))
                         & masky)
                wif = rr * float(WCN) + rc - float(WR0 * WCN + WC0)
                widx = jnp.where(inwin, wif, 0.0).astype(jnp.int32)
                plsc.addupdate_scatter(wbuf, [widx], ones16, mask=inwin)
                rany = rany | (masky & (~inwin))
            return rany

        rany = lax.fori_loop(0, GROUPS // 8, _row,
                             jnp.zeros((16,), jnp.bool_))
        fire = jnp.any(rany)

        @pl.when(fire)
        def _slow():
            def _srow(r, carry):
                for j in range(8):
                    o = r * 128 + j * 16
                    xs = xv[pl.ds(o, 16)]
                    ys = yv[pl.ds(o, 16)]
                    zs = zv[pl.ds(o, 16)]
                    rr, rc = _cell_rows(xs, zs)
                    masky = (ys < 0.0) & (ys > -1.0)
                    inwin = ((rr >= float(WR0)) & (rr < float(WR0 + WRN))
                             & (rc >= float(WC0)) & (rc < float(WC0 + WCN)))
                    inb = ((rr >= 0.0) & (rr <= 399.0)
                           & (rc >= 0.0) & (rc <= 399.0))
                    rest = masky & inb & (~inwin)
                    idxf = jnp.where(rest, rr * 400.0 + rc, float(DUMP))
                    idxrow[0, pl.ds(j * 16, 16)] = idxf.astype(jnp.int32)
                pltpu.sync_copy(onesv, hist.at[idxrow.at[0]], add=True)
                return carry

            lax.fori_loop(0, GROUPS // 8, _srow, None)

    def _fire(k, bufs, sem):
        xv, yv, zv = bufs
        off = base + k * CHUNK
        pltpu.async_copy(xh.at[pl.ds(off, CHUNK)], xv, sem)
        pltpu.async_copy(yh.at[pl.ds(off, CHUNK)], yv, sem)
        pltpu.async_copy(zh.at[pl.ds(off, CHUNK)], zv, sem)

    def _wait(bufs, sem):
        xv, yv, zv = bufs
        pltpu.make_async_copy(xh.at[pl.ds(0, CHUNK)], xv, sem).wait()
        pltpu.make_async_copy(xh.at[pl.ds(0, CHUNK)], yv, sem).wait()
        pltpu.make_async_copy(xh.at[pl.ds(0, CHUNK)], zv, sem).wait()

    bufs0 = (xv0, yv0, zv0)
    bufs1 = (xv1, yv1, zv1)
    _fire(0, bufs0, sem0)
    _fire(1, bufs1, sem1)

    def _pair(kk, carry):
        k0 = kk * 2
        _wait(bufs0, sem0)
        _process(bufs0)

        @pl.when(k0 + 2 < N_CHUNKS)
        def _f0():
            _fire(k0 + 2, bufs0, sem0)

        _wait(bufs1, sem1)
        _process(bufs1)

        @pl.when(k0 + 3 < N_CHUNKS)
        def _f1():
            _fire(k0 + 3, bufs1, sem1)

        return carry

    lax.fori_loop(0, N_CHUNKS // 2, _pair, None)
    plsc.subcore_barrier()

    pltpu.sync_copy(wbuf, outw.at[pl.ds(wid * WSZ, WSZ)])
    pltpu.sync_copy(hist.at[pl.ds(s * OSLICE, OSLICE)],
                    obuf.at[pl.ds(0, OSLICE)])
    pltpu.sync_copy(obuf.at[pl.ds(0, OSLICE)],
                    outg.at[pl.ds(c * N_BINS + s * OSLICE, OSLICE)])


_sc_hist = pl.kernel(
    _sc_body,
    out_type=(jax.ShapeDtypeStruct((2 * N_BINS,), jnp.float32),
              jax.ShapeDtypeStruct((NW * WSZ,), jnp.float32)),
    mesh=plsc.VectorSubcoreMesh(core_axis_name="c", subcore_axis_name="s"),
    compiler_params=pltpu.CompilerParams(needs_layout_passes=False,
                                         use_tc_tiling_on_sc=False),
    scratch_types=[
        pltpu.VMEM((CHUNK,), jnp.float32),
        pltpu.VMEM((CHUNK,), jnp.float32),
        pltpu.VMEM((CHUNK,), jnp.float32),
        pltpu.VMEM((CHUNK,), jnp.float32),
        pltpu.VMEM((CHUNK,), jnp.float32),
        pltpu.VMEM((CHUNK,), jnp.float32),
        pltpu.VMEM((1, 128), jnp.int32),
        pltpu.VMEM((128,), jnp.float32),
        pltpu.VMEM((WSZ,), jnp.float32),
        pltpu.VMEM((ZSLICE,), jnp.float32),
        pltpu.VMEM_SHARED((BINS_PAD,), jnp.float32),
        pltpu.SemaphoreType.DMA,
        pltpu.SemaphoreType.DMA,
    ],
)


def _final_body(g_ref, w_ref, o_ref):
    o_ref[...] = g_ref[0] + g_ref[1]
    wsum = jnp.sum(w_ref[...], axis=0)
    o_ref[pl.ds(WR0, WRN), pl.ds(WC0, WCN)] = (
        o_ref[pl.ds(WR0, WRN), pl.ds(WC0, WCN)] + wsum)


def kernel(sparse_depth):
    n = sparse_depth.shape[0]
    pts = jnp.pad(sparse_depth.T, ((0, 0), (0, NP - n)))
    og, ow = _sc_hist(pts[0], pts[1], pts[2])
    g = og.reshape(2, MAP_CELLS, MAP_CELLS)
    w = ow.reshape(NW, WRN, WCN)
    return pl.pallas_call(
        _final_body,
        out_shape=jax.ShapeDtypeStruct((MAP_CELLS, MAP_CELLS), jnp.float32),
    )(g, w)
